# baseline (speedup 1.0000x reference)
import jax
import jax.numpy as jnp
from jax import lax
from jax.experimental import pallas as pl
from jax.experimental.pallas import tpu as pltpu

_T = 200
_V = 1000
_D = 128
_SAMPLING_RATIO = 0.2


def _tc_body(lens_ref, ig_ref, dec_ref, ys_ref, pa_ref, rc_ref, rr_ref, w_ref,
             out1_ref, out2_ref, out3_ref, tg_ref, tn_ref, ts_ref, tr_ref):
    b = pl.program_id(0)
    L = lens_ref[b]
    ig = ig_ref[0]
    d = dec_ref[0]
    ys = ys_ref[0]
    pa = pa_ref[0]
    rcol = rc_ref[0]
    rrow = rr_ref[0]

    viota = lax.broadcasted_iota(jnp.int32, (_T, _V), 1)
    mx = jnp.max(d, axis=1, keepdims=True)
    pred = jnp.min(jnp.where(d == mx, viota, _V), axis=1, keepdims=True)

    not_ignore = ys != ig
    same = (ys == pred) & not_ignore
    same_num = jnp.sum(same.astype(jnp.int32))
    eff = jnp.maximum(
        ((L.astype(jnp.float32) - same_num.astype(jnp.float32))
         * _SAMPLING_RATIO).astype(jnp.int32), 0)

    tio = lax.broadcasted_iota(jnp.int32, (_T, _T), 0)
    uio = lax.broadcasted_iota(jnp.int32, (_T, _T), 1)
    gt = (rrow > rcol) | ((rrow == rcol) & (uio < tio))
    validu = uio < L
    rank = jnp.sum((gt & validu).astype(jnp.int32), axis=1, keepdims=True)

    t2 = lax.broadcasted_iota(jnp.int32, (_T, 1), 0)
    tgt = t2 < L
    imask = (rank < eff) & tgt & not_ignore

    idx = jnp.where(tgt, ys, 0)
    onehot = (viota == idx).astype(jnp.float32)
    emb = lax.dot_general(onehot, w_ref[...], (((1,), (0,)), ((), ())),
                          preferred_element_type=jnp.float32,
                          precision=lax.Precision.HIGHEST)

    tgtf = tgt.astype(jnp.float32)
    out1_ref[0] = jnp.where(imask, emb, pa) * tgtf
    out2_ref[0] = emb * tgtf
    out3_ref[0] = pa * tgtf
    tg_ref[0] = tgt.astype(jnp.int32)

    num = jnp.sum(not_ignore.astype(jnp.int32))

    @pl.when(b == 0)
    def _init():
        tn_ref[0, 0] = 0
        ts_ref[0, 0] = 0
        tr_ref[0, 0] = 0

    tn_ref[0, 0] += num
    ts_ref[0, 0] += same_num
    tr_ref[0, 0] += eff


def kernel(decoder_out, ys_pad, ys_pad_lens, pred_acoustic_embeds, ignore_id, W):
    B, T = ys_pad.shape
    r = jax.random.uniform(jax.random.key(123), (B, T))
    rcol = r.reshape(B, T, 1)
    rrow = r.reshape(B, 1, T)
    ys3 = ys_pad.astype(jnp.int32).reshape(B, T, 1)
    lens = ys_pad_lens.astype(jnp.int32)
    ig = jnp.asarray(ignore_id, jnp.int32).reshape(1)

    out_shapes = (
        jax.ShapeDtypeStruct((B, T, _D), jnp.float32),
        jax.ShapeDtypeStruct((B, T, _D), jnp.float32),
        jax.ShapeDtypeStruct((B, T, _D), jnp.float32),
        jax.ShapeDtypeStruct((B, T, 1), jnp.int32),
        jax.ShapeDtypeStruct((1, 1), jnp.int32),
        jax.ShapeDtypeStruct((1, 1), jnp.int32),
        jax.ShapeDtypeStruct((1, 1), jnp.int32),
    )
    grid = (B,)
    smem = pltpu.SMEM
    in_specs = [
        pl.BlockSpec(memory_space=smem),
        pl.BlockSpec(memory_space=smem),
        pl.BlockSpec((1, T, _V), lambda b: (b, 0, 0)),
        pl.BlockSpec((1, T, 1), lambda b: (b, 0, 0)),
        pl.BlockSpec((1, T, _D), lambda b: (b, 0, 0)),
        pl.BlockSpec((1, T, 1), lambda b: (b, 0, 0)),
        pl.BlockSpec((1, 1, T), lambda b: (b, 0, 0)),
        pl.BlockSpec((_V, _D), lambda b: (0, 0)),
    ]
    out_specs = [
        pl.BlockSpec((1, T, _D), lambda b: (b, 0, 0)),
        pl.BlockSpec((1, T, _D), lambda b: (b, 0, 0)),
        pl.BlockSpec((1, T, _D), lambda b: (b, 0, 0)),
        pl.BlockSpec((1, T, 1), lambda b: (b, 0, 0)),
        pl.BlockSpec((1, 1), lambda b: (0, 0), memory_space=smem),
        pl.BlockSpec((1, 1), lambda b: (0, 0), memory_space=smem),
        pl.BlockSpec((1, 1), lambda b: (0, 0), memory_space=smem),
    ]
    o1, o2, o3, tg, tn, ts, tr = pl.pallas_call(
        _tc_body,
        grid=grid,
        in_specs=in_specs,
        out_specs=out_specs,
        out_shape=out_shapes,
    )(lens, ig, decoder_out, ys3, pred_acoustic_embeds, rcol, rrow, W)

    tgt3 = tg.astype(jnp.bool_)
    return (o1, o2, o3, tgt3,
            tn.reshape(()), ts.reshape(()), tr.reshape(()))
